# Initial kernel scaffold; baseline (speedup 1.0000x reference)
import jax
import jax.numpy as jnp
from jax.experimental import pallas as pl


def kernel(embedding, node_edges, edge_nodes, W_p, b_p, W_q, b_q, W_p2a, b_p2a, W_p2b, b_p2b, W_a, b_a):
    raise NotImplementedError("write your pallas kernel here")


# speedup vs baseline: 2.3034x; 2.3034x over previous
import functools
import math

import jax
import jax.numpy as jnp
from jax import lax
from jax.experimental import pallas as pl
from jax.experimental.pallas import tpu as pltpu
from jax.experimental.pallas import tpu_sc as plsc

_NC, _NS = 2, 16
_NW = _NC * _NS


def _tc_proj_body(x_ref, wa_ref, ba_ref, wp_ref, bp_ref, w2a_ref, b2a_ref,
                  w2b_ref, b2b_ref, res_ref, tv_ref, e2_ref):
    f32 = jnp.float32
    x = x_ref[...]
    res_ref[...] = jnp.maximum(
        jnp.dot(x, wa_ref[...], preferred_element_type=f32) + ba_ref[...], 0.0)
    tv_ref[...] = jnp.dot(x, wp_ref[...], preferred_element_type=f32) + bp_ref[...]
    h = jnp.maximum(
        jnp.dot(x, w2a_ref[...], preferred_element_type=f32) + b2a_ref[...], 0.0)
    e2_ref[...] = jnp.dot(h, w2b_ref[...], preferred_element_type=f32) + b2b_ref[...]


def _tc_out_body(s1_ref, s2_ref, res_ref, wq_ref, bq_ref, o_ref):
    t = (jnp.dot(s1_ref[...], wq_ref[...], preferred_element_type=jnp.float32)
         + bq_ref[...] + s2_ref[...])
    o_ref[...] = jnp.maximum(t, 0.0) + res_ref[...]


def _edge_body(enf_ref, e2t_ref, er_ref, idx_v, v2_v, s_v, sem, *, ec, iters, inv_d):
    wid = lax.axis_index("s") * _NC + lax.axis_index("c")

    def it_body(it, _):
        ebase = (wid * iters + it) * ec
        pltpu.sync_copy(enf_ref.at[pl.ds(ebase * 4, ec * 4)], idx_v)
        for j in range(ec * 4 // 128):
            pltpu.async_copy(
                e2t_ref.at[idx_v.at[pl.ds(j * 128, 128)]],
                v2_v.at[pl.ds(j * 128, 128)], sem).wait()

        def t_body(t, _):
            for s in range(8):
                sl = pl.ds(s * 16, 16)
                acc = (v2_v[4 * t, sl] + v2_v[4 * t + 1, sl]
                       + v2_v[4 * t + 2, sl] + v2_v[4 * t + 3, sl])
                s_v[t, sl] = jnp.maximum(acc, 0.0) * inv_d
            return 0

        lax.fori_loop(0, ec, t_body, 0)
        pltpu.sync_copy(s_v, er_ref.at[pl.ds(ebase, ec)])
        return 0

    lax.fori_loop(0, iters, it_body, 0)


def _node_body(nef_ref, en0_ref, en1_ref, en2_ref, en3_ref, tv_ref, er_ref,
               s1_ref, s2_ref,
               ne_v, mi_v, vb_v, e2b_v, s1b_v, s2b_v, sem,
               *, npc, iters, d_deg, two_c):
    wid = lax.axis_index("s") * _NC + lax.axis_index("c")
    npairs = npc * d_deg
    en_refs = (en0_ref, en1_ref, en2_ref, en3_ref)

    def it_body(it, _):
        nbase = (wid * iters + it) * npc
        pbase = nbase * d_deg
        pltpu.sync_copy(nef_ref.at[pl.ds(pbase, npairs)], ne_v)
        for j in range(npairs // 128):
            sl = pl.ds(j * 128, 128)
            pltpu.async_copy(er_ref.at[ne_v.at[sl]], e2b_v.at[sl], sem).wait()
            for i in range(4):
                pltpu.async_copy(en_refs[i].at[ne_v.at[sl]], mi_v.at[i, sl],
                                 sem).wait()
        for i in range(4):
            for j in range(npairs // 128):
                sl = pl.ds(j * 128, 128)
                pltpu.async_copy(tv_ref.at[mi_v.at[i, sl]],
                                 vb_v.at[pl.ds(i * npairs + j * 128, 128)],
                                 sem).wait()

        def nl_body(nl, _):
            n_abs = nbase + nl
            pb = nl * d_deg
            mrows = [mi_v[i, pl.ds(pb, 16)] for i in range(4)]
            z = jnp.zeros((16,), jnp.float32)
            acc1 = [z] * 4
            acc2 = [z] * 8
            for dd in range(d_deg):
                p = pb + dd
                neqs = [mrows[i][dd] != n_abs for i in range(4)]
                for s in range(4):
                    sl = pl.ds(s * 16, 16)
                    pr = None
                    for i in range(4):
                        vv = jnp.where(neqs[i], vb_v[i * npairs + p, sl], 1.0)
                        pr = vv if pr is None else pr * vv
                    th = 1.0 - 2.0 / (jnp.exp(pr * two_c) + 1.0)
                    acc1[s] = acc1[s] + th
                for s in range(8):
                    acc2[s] = acc2[s] + e2b_v[p, pl.ds(s * 16, 16)]
            for s in range(4):
                s1b_v[nl, pl.ds(s * 16, 16)] = acc1[s]
                s1b_v[nl, pl.ds((4 + s) * 16, 16)] = z
            for s in range(8):
                s2b_v[nl, pl.ds(s * 16, 16)] = acc2[s]
            return 0

        lax.fori_loop(0, npc, nl_body, 0)
        pltpu.sync_copy(s1b_v, s1_ref.at[pl.ds(nbase, npc)])
        pltpu.sync_copy(s2b_v, s2_ref.at[pl.ds(nbase, npc)])
        return 0

    lax.fori_loop(0, iters, it_body, 0)


def kernel(embedding, node_edges, edge_nodes, W_p, b_p, W_q, b_q,
           W_p2a, b_p2a, W_p2b, b_p2b, W_a, b_a):
    f32 = jnp.float32
    n, feat = embedding.shape
    d_deg = node_edges.shape[1]
    e, k = edge_nodes.shape
    rank = W_p.shape[1]
    out = W_q.shape[1]
    hid = W_p2a.shape[1]

    num = float(d_deg) ** (1.0 / float(k))
    two_c = 2.0 * num / float(math.factorial(k - 1))
    inv_d = 1.0 / float(d_deg)

    rank_p = 128
    npc = 8
    n_pad = -(-n // (_NW * npc)) * (_NW * npc)
    ec = 128
    e_pad = -(-e // (_NW * ec)) * (_NW * ec)

    ne32 = node_edges.astype(jnp.int32)
    en32 = edge_nodes.astype(jnp.int32)
    x_pad = jnp.zeros((n_pad, feat), f32).at[:n].set(embedding.astype(f32))
    nef = jnp.zeros((n_pad, d_deg), jnp.int32).at[:n].set(ne32).reshape(-1)
    enf = jnp.zeros((e_pad, k), jnp.int32).at[:e].set(en32).reshape(-1)

    wa_x = W_a[:feat]
    ba_full = (b_a + W_a[feat]).reshape(1, out)
    wp_x = jnp.zeros((feat, rank_p), f32).at[:, :rank].set(num * W_p[:feat])
    bp_full = jnp.zeros((1, rank_p), f32).at[0, :rank].set(num * (b_p + W_p[feat]))
    w2a_x = W_p2a[:feat]
    b2a_full = (b_p2a + W_p2a[feat]).reshape(1, hid)
    b2b_full = b_p2b.reshape(1, out)
    wq_s = jnp.zeros((rank_p, out), f32).at[:rank].set(W_q * inv_d)
    bq_full = b_q.reshape(1, out)

    br = 512
    grid = (n_pad // br,)
    wspec = lambda a: pl.BlockSpec(a.shape, lambda i: (0, 0))
    res, tv, e2t = pl.pallas_call(
        _tc_proj_body,
        grid=grid,
        in_specs=[pl.BlockSpec((br, feat), lambda i: (i, 0)),
                  wspec(wa_x), wspec(ba_full), wspec(wp_x), wspec(bp_full),
                  wspec(w2a_x), wspec(b2a_full), wspec(w2b := W_p2b), wspec(b2b_full)],
        out_specs=[pl.BlockSpec((br, out), lambda i: (i, 0)),
                   pl.BlockSpec((br, rank_p), lambda i: (i, 0)),
                   pl.BlockSpec((br, out), lambda i: (i, 0))],
        out_shape=[jax.ShapeDtypeStruct((n_pad, out), f32),
                   jax.ShapeDtypeStruct((n_pad, rank_p), f32),
                   jax.ShapeDtypeStruct((n_pad, out), f32)],
    )(x_pad, wa_x, ba_full, wp_x, bp_full, w2a_x, b2a_full, w2b, b2b_full)

    mesh = plsc.VectorSubcoreMesh(core_axis_name="c", subcore_axis_name="s")

    e_iters = e_pad // (_NW * ec)
    er = pl.kernel(
        functools.partial(_edge_body, ec=ec, iters=e_iters, inv_d=inv_d),
        out_type=jax.ShapeDtypeStruct((e_pad, out), f32),
        mesh=mesh,
        scratch_types=[
            pltpu.VMEM((ec * 4,), jnp.int32),
            pltpu.VMEM((ec * 4, out), f32),
            pltpu.VMEM((ec, out), f32),
            pltpu.SemaphoreType.DMA,
        ],
    )(enf, e2t)

    n_iters = n_pad // (_NW * npc)
    npairs = npc * d_deg
    s1, s2 = pl.kernel(
        functools.partial(_node_body, npc=npc, iters=n_iters, d_deg=d_deg,
                          two_c=two_c),
        out_type=(jax.ShapeDtypeStruct((n_pad, rank_p), f32),
                  jax.ShapeDtypeStruct((n_pad, out), f32)),
        mesh=mesh,
        scratch_types=[
            pltpu.VMEM((npairs,), jnp.int32),
            pltpu.VMEM((4, npairs), jnp.int32),
            pltpu.VMEM((npairs * 4, rank_p), f32),
            pltpu.VMEM((npairs, out), f32),
            pltpu.VMEM((npc, rank_p), f32),
            pltpu.VMEM((npc, out), f32),
            pltpu.SemaphoreType.DMA,
        ],
    )(nef, en32[:, 0], en32[:, 1], en32[:, 2], en32[:, 3], tv, er)

    o = pl.pallas_call(
        _tc_out_body,
        grid=grid,
        in_specs=[pl.BlockSpec((br, rank_p), lambda i: (i, 0)),
                  pl.BlockSpec((br, out), lambda i: (i, 0)),
                  pl.BlockSpec((br, out), lambda i: (i, 0)),
                  wspec(wq_s), wspec(bq_full)],
        out_specs=pl.BlockSpec((br, out), lambda i: (i, 0)),
        out_shape=jax.ShapeDtypeStruct((n_pad, out), f32),
    )(s1, s2, res, wq_s, bq_full)

    return o[:n]


# speedup vs baseline: 3.3013x; 1.4332x over previous
import functools
import math

import jax
import jax.numpy as jnp
from jax import lax
from jax.experimental import pallas as pl
from jax.experimental.pallas import tpu as pltpu
from jax.experimental.pallas import tpu_sc as plsc

_NC, _NS = 2, 16
_NW = _NC * _NS


def _tc_proj_body(x_ref, wa_ref, ba_ref, wp_ref, bp_ref, w2a_ref, b2a_ref,
                  w2b_ref, b2b_ref, res_ref, tv_ref, e2_ref):
    f32 = jnp.float32
    x = x_ref[...]
    res_ref[...] = jnp.maximum(
        jnp.dot(x, wa_ref[...], preferred_element_type=f32) + ba_ref[...], 0.0)
    tv_ref[...] = jnp.dot(x, wp_ref[...], preferred_element_type=f32) + bp_ref[...]
    h = jnp.maximum(
        jnp.dot(x, w2a_ref[...], preferred_element_type=f32) + b2a_ref[...], 0.0)
    e2_ref[...] = jnp.dot(h, w2b_ref[...], preferred_element_type=f32) + b2b_ref[...]


def _tc_out_body(s1_ref, s2_ref, res_ref, wq_ref, bq_ref, o_ref):
    t = (jnp.dot(s1_ref[...], wq_ref[...], preferred_element_type=jnp.float32)
         + bq_ref[...] + s2_ref[...])
    o_ref[...] = jnp.maximum(t, 0.0) + res_ref[...]


def _edge_body(enf_ref, e2t_ref, er_ref, idx_v, v2_v, s_v, sem, *, ec, iters, inv_d):
    wid = lax.axis_index("s") * _NC + lax.axis_index("c")

    def it_body(it, _):
        ebase = (wid * iters + it) * ec
        pltpu.sync_copy(enf_ref.at[pl.ds(ebase * 4, ec * 4)], idx_v)
        cps = [pltpu.async_copy(
                   e2t_ref.at[idx_v.at[pl.ds(j * 128, 128)]],
                   v2_v.at[pl.ds(j * 128, 128)], sem)
               for j in range(ec * 4 // 128)]
        for c in cps:
            c.wait()

        def t_body(t, _):
            for s in range(8):
                sl = pl.ds(s * 16, 16)
                acc = (v2_v[4 * t, sl] + v2_v[4 * t + 1, sl]
                       + v2_v[4 * t + 2, sl] + v2_v[4 * t + 3, sl])
                s_v[t, sl] = jnp.maximum(acc, 0.0) * inv_d
            return 0

        lax.fori_loop(0, ec, t_body, 0)
        pltpu.sync_copy(s_v, er_ref.at[pl.ds(ebase, ec)])
        return 0

    lax.fori_loop(0, iters, it_body, 0)


def _node_body(nef_ref, en0_ref, en1_ref, en2_ref, en3_ref, tvb_ref, er_ref,
               s1_ref, s2_ref,
               ne_v, mi_v, mi2_v, vb_v, e2b_v, s1b_v, s2b_v, sem, sem2,
               *, npc, iters, d_deg, two_c, ones_row):
    wid = lax.axis_index("s") * _NC + lax.axis_index("c")
    npairs = npc * d_deg
    en_refs = (en0_ref, en1_ref, en2_ref, en3_ref)

    def it_body(it, _):
        nbase = (wid * iters + it) * npc
        pbase = nbase * d_deg
        pltpu.sync_copy(nef_ref.at[pl.ds(pbase, npairs)], ne_v)
        c_er = pltpu.async_copy(er_ref.at[ne_v.at[pl.ds(0, npairs)]],
                                e2b_v, sem2)
        c_mi = [pltpu.async_copy(en_refs[i].at[ne_v.at[pl.ds(0, npairs)]],
                                 mi_v.at[i], sem) for i in range(4)]
        for c in c_mi:
            c.wait()
        for i in range(4):
            for nl in range(npc):
                sl = pl.ds(nl * d_deg, d_deg)
                v = mi_v[i, sl]
                mi2_v[i, sl] = jnp.where(v == nbase + nl, ones_row, v)
        c_vb = [pltpu.async_copy(tvb_ref.at[mi2_v.at[i]],
                                 vb_v.at[pl.ds(i * npairs, npairs)], sem)
                for i in range(4)]
        for c in c_vb:
            c.wait()
        c_er.wait()

        def nl_body(nl, _):
            pb = nl * d_deg
            z = jnp.zeros((16,), jnp.float32)
            acc1 = [z] * 4
            acc2 = [z] * 8
            for dd in range(d_deg):
                p = pb + dd
                for s in range(4):
                    sl = pl.ds(s * 16, 16)
                    pr = (vb_v[p, sl] * vb_v[npairs + p, sl]
                          * vb_v[2 * npairs + p, sl] * vb_v[3 * npairs + p, sl])
                    th = 1.0 - 2.0 / (jnp.exp(pr * two_c) + 1.0)
                    acc1[s] = acc1[s] + th
                for s in range(8):
                    acc2[s] = acc2[s] + e2b_v[p, pl.ds(s * 16, 16)]
            for s in range(4):
                s1b_v[nl, pl.ds(s * 16, 16)] = acc1[s]
            for s in range(4, 8):
                s1b_v[nl, pl.ds(s * 16, 16)] = z
            for s in range(8):
                s2b_v[nl, pl.ds(s * 16, 16)] = acc2[s]
            return 0

        lax.fori_loop(0, npc, nl_body, 0)
        pltpu.sync_copy(s1b_v, s1_ref.at[pl.ds(nbase, npc)])
        pltpu.sync_copy(s2b_v, s2_ref.at[pl.ds(nbase, npc)])
        return 0

    lax.fori_loop(0, iters, it_body, 0)


def kernel(embedding, node_edges, edge_nodes, W_p, b_p, W_q, b_q,
           W_p2a, b_p2a, W_p2b, b_p2b, W_a, b_a):
    f32 = jnp.float32
    n, feat = embedding.shape
    d_deg = node_edges.shape[1]
    e, k = edge_nodes.shape
    rank = W_p.shape[1]
    out = W_q.shape[1]
    hid = W_p2a.shape[1]

    num = float(d_deg) ** (1.0 / float(k))
    two_c = 2.0 * num / float(math.factorial(k - 1))
    inv_d = 1.0 / float(d_deg)

    rank_p = 128
    npc = 8
    n_pad = -(-n // (_NW * npc)) * (_NW * npc)
    ec = 128
    e_pad = -(-e // (_NW * ec)) * (_NW * ec)

    ne32 = node_edges.astype(jnp.int32)
    en32 = edge_nodes.astype(jnp.int32)
    x_pad = jnp.zeros((n_pad, feat), f32).at[:n].set(embedding.astype(f32))
    nef = jnp.zeros((n_pad, d_deg), jnp.int32).at[:n].set(ne32).reshape(-1)
    enf = jnp.zeros((e_pad, k), jnp.int32).at[:e].set(en32).reshape(-1)

    wa_x = W_a[:feat]
    ba_full = (b_a + W_a[feat]).reshape(1, out)
    wp_x = jnp.zeros((feat, rank_p), f32).at[:, :rank].set(num * W_p[:feat])
    bp_full = jnp.zeros((1, rank_p), f32).at[0, :rank].set(num * (b_p + W_p[feat]))
    w2a_x = W_p2a[:feat]
    b2a_full = (b_p2a + W_p2a[feat]).reshape(1, hid)
    b2b_full = b_p2b.reshape(1, out)
    wq_s = jnp.zeros((rank_p, out), f32).at[:rank].set(W_q * inv_d)
    bq_full = b_q.reshape(1, out)

    br = 512
    grid = (n_pad // br,)
    wspec = lambda a: pl.BlockSpec(a.shape, lambda i: (0, 0))
    res, tv, e2t = pl.pallas_call(
        _tc_proj_body,
        grid=grid,
        in_specs=[pl.BlockSpec((br, feat), lambda i: (i, 0)),
                  wspec(wa_x), wspec(ba_full), wspec(wp_x), wspec(bp_full),
                  wspec(w2a_x), wspec(b2a_full), wspec(w2b := W_p2b), wspec(b2b_full)],
        out_specs=[pl.BlockSpec((br, out), lambda i: (i, 0)),
                   pl.BlockSpec((br, rank_p), lambda i: (i, 0)),
                   pl.BlockSpec((br, out), lambda i: (i, 0))],
        out_shape=[jax.ShapeDtypeStruct((n_pad, out), f32),
                   jax.ShapeDtypeStruct((n_pad, rank_p), f32),
                   jax.ShapeDtypeStruct((n_pad, out), f32)],
    )(x_pad, wa_x, ba_full, wp_x, bp_full, w2a_x, b2a_full, w2b, b2b_full)

    tvb = jnp.concatenate([tv, jnp.ones((8, rank_p), f32)], axis=0)

    mesh = plsc.VectorSubcoreMesh(core_axis_name="c", subcore_axis_name="s")

    e_iters = e_pad // (_NW * ec)
    er = pl.kernel(
        functools.partial(_edge_body, ec=ec, iters=e_iters, inv_d=inv_d),
        out_type=jax.ShapeDtypeStruct((e_pad, out), f32),
        mesh=mesh,
        scratch_types=[
            pltpu.VMEM((ec * 4,), jnp.int32),
            pltpu.VMEM((ec * 4, out), f32),
            pltpu.VMEM((ec, out), f32),
            pltpu.SemaphoreType.DMA,
        ],
    )(enf, e2t)

    n_iters = n_pad // (_NW * npc)
    npairs = npc * d_deg
    s1, s2 = pl.kernel(
        functools.partial(_node_body, npc=npc, iters=n_iters, d_deg=d_deg,
                          two_c=two_c, ones_row=n_pad),
        out_type=(jax.ShapeDtypeStruct((n_pad, rank_p), f32),
                  jax.ShapeDtypeStruct((n_pad, out), f32)),
        mesh=mesh,
        scratch_types=[
            pltpu.VMEM((npairs,), jnp.int32),
            pltpu.VMEM((4, npairs), jnp.int32),
            pltpu.VMEM((4, npairs), jnp.int32),
            pltpu.VMEM((npairs * 4, rank_p), f32),
            pltpu.VMEM((npairs, out), f32),
            pltpu.VMEM((npc, rank_p), f32),
            pltpu.VMEM((npc, out), f32),
            pltpu.SemaphoreType.DMA,
            pltpu.SemaphoreType.DMA,
        ],
    )(nef, en32[:, 0], en32[:, 1], en32[:, 2], en32[:, 3], tvb, er)

    o = pl.pallas_call(
        _tc_out_body,
        grid=grid,
        in_specs=[pl.BlockSpec((br, rank_p), lambda i: (i, 0)),
                  pl.BlockSpec((br, out), lambda i: (i, 0)),
                  pl.BlockSpec((br, out), lambda i: (i, 0)),
                  wspec(wq_s), wspec(bq_full)],
        out_specs=pl.BlockSpec((br, out), lambda i: (i, 0)),
        out_shape=jax.ShapeDtypeStruct((n_pad, out), f32),
    )(s1, s2, res, wq_s, bq_full)

    return o[:n]


# speedup vs baseline: 3.7382x; 1.1324x over previous
import functools
import math

import jax
import jax.numpy as jnp
from jax import lax
from jax.experimental import pallas as pl
from jax.experimental.pallas import tpu as pltpu
from jax.experimental.pallas import tpu_sc as plsc

_NC, _NS = 2, 16
_NW = _NC * _NS


def _tc_proj_body(x_ref, wa_ref, ba_ref, wp_ref, bp_ref, w2a_ref, b2a_ref,
                  w2b_ref, b2b_ref, res_ref, tv_ref, e2_ref):
    f32 = jnp.float32
    x = x_ref[...]
    res_ref[...] = jnp.maximum(
        jnp.dot(x, wa_ref[...], preferred_element_type=f32) + ba_ref[...], 0.0)
    tv_ref[...] = jnp.dot(x, wp_ref[...], preferred_element_type=f32) + bp_ref[...]
    h = jnp.maximum(
        jnp.dot(x, w2a_ref[...], preferred_element_type=f32) + b2a_ref[...], 0.0)
    e2_ref[...] = jnp.dot(h, w2b_ref[...], preferred_element_type=f32) + b2b_ref[...]


def _tc_out_body(s1_ref, s2_ref, res_ref, wq_ref, bq_ref, o_ref):
    t = (jnp.dot(s1_ref[...], wq_ref[...], preferred_element_type=jnp.float32)
         + bq_ref[...] + s2_ref[...])
    o_ref[...] = jnp.maximum(t, 0.0) + res_ref[...]


def _edge_body(enf_ref, tvb_ref, e2t_ref, ct_ref, idx_v, rows_v, ct_v, sem,
               *, ec, iters, inv_d, two_c):
    wid = lax.axis_index("s") * _NC + lax.axis_index("c")

    def it_body(it, _):
        ebase = (wid * iters + it) * ec
        pltpu.sync_copy(enf_ref.at[pl.ds(ebase * 4, ec * 4)], idx_v)
        cps = [pltpu.async_copy(
                   tvb_ref.at[idx_v.at[pl.ds(j * 128, 128)]],
                   rows_v.at[pl.ds(j * 128, 128)], sem)
               for j in range(ec * 4 // 128)]
        for c in cps:
            c.wait()

        def t1_body(t, _):
            for s in range(4):
                sl = pl.ds(s * 16, 16)
                pr = (rows_v[4 * t, sl] * rows_v[4 * t + 1, sl]
                      * rows_v[4 * t + 2, sl] * rows_v[4 * t + 3, sl])
                ct_v[t, sl] = 1.0 - 2.0 / (jnp.exp(pr * two_c) + 1.0)
            return 0

        lax.fori_loop(0, ec, t1_body, 0)
        cps = [pltpu.async_copy(
                   e2t_ref.at[idx_v.at[pl.ds(j * 128, 128)]],
                   rows_v.at[pl.ds(j * 128, 128)], sem)
               for j in range(ec * 4 // 128)]
        for c in cps:
            c.wait()

        def t2_body(t, _):
            for s in range(8):
                sl = pl.ds(s * 16, 16)
                acc = (rows_v[4 * t, sl] + rows_v[4 * t + 1, sl]
                       + rows_v[4 * t + 2, sl] + rows_v[4 * t + 3, sl])
                ct_v[t, pl.ds(64 + s * 16, 16)] = jnp.maximum(acc, 0.0) * inv_d
            return 0

        lax.fori_loop(0, ec, t2_body, 0)
        pltpu.sync_copy(ct_v, ct_ref.at[pl.ds(ebase, ec)])
        return 0

    lax.fori_loop(0, iters, it_body, 0)


def _node_body(nef_ref, fl_ref, en0_ref, en1_ref, en2_ref, en3_ref, tvb_ref,
               ct_ref, s1_ref, s2_ref,
               ne_v, mi_v, mi2_v, ct_b, vb_v, s1b_v, s2b_v, flag_v, sem, sem2,
               *, npc, iters, d_deg, two_c, ones_row):
    wid = lax.axis_index("s") * _NC + lax.axis_index("c")
    npairs = npc * d_deg
    en_refs = (en0_ref, en1_ref, en2_ref, en3_ref)
    pltpu.sync_copy(fl_ref.at[pl.ds(wid * 128, 128)], flag_v)

    def it_body(it, _):
        chunk = wid * iters + it
        nbase = chunk * npc
        pbase = nbase * d_deg
        pltpu.sync_copy(nef_ref.at[pl.ds(pbase, npairs)], ne_v)
        c_ct = pltpu.async_copy(ct_ref.at[ne_v.at[pl.ds(0, npairs)]],
                                ct_b, sem2)
        cnt = flag_v[pl.ds(it, 16)][0]
        c_ct.wait()

        def nl_fast(nl, _):
            pb = nl * d_deg
            z = jnp.zeros((16,), jnp.float32)
            acc1 = [z] * 4
            acc2 = [z] * 8
            for dd in range(d_deg):
                p = pb + dd
                for s in range(4):
                    acc1[s] = acc1[s] + ct_b[p, pl.ds(s * 16, 16)]
                for s in range(8):
                    acc2[s] = acc2[s] + ct_b[p, pl.ds(64 + s * 16, 16)]
            for s in range(4):
                s1b_v[nl, pl.ds(s * 16, 16)] = acc1[s]
            for s in range(8):
                s2b_v[nl, pl.ds(s * 16, 16)] = acc2[s]
            return 0

        lax.fori_loop(0, npc, nl_fast, 0)

        @pl.when(cnt != 0)
        def _():
            c_mi = [pltpu.async_copy(en_refs[i].at[ne_v.at[pl.ds(0, npairs)]],
                                     mi_v.at[i], sem) for i in range(4)]
            for c in c_mi:
                c.wait()
            for i in range(4):
                for nl in range(npc):
                    sl = pl.ds(nl * d_deg, d_deg)
                    v = mi_v[i, sl]
                    mi2_v[i, sl] = jnp.where(v == nbase + nl, ones_row, v)
            c_vb = [pltpu.async_copy(tvb_ref.at[mi2_v.at[i]],
                                     vb_v.at[pl.ds(i * npairs, npairs)], sem)
                    for i in range(4)]
            for c in c_vb:
                c.wait()

            def nl_slow(nl, _):
                pb = nl * d_deg
                z = jnp.zeros((16,), jnp.float32)
                acc1 = [z] * 4
                for dd in range(d_deg):
                    p = pb + dd
                    for s in range(4):
                        sl = pl.ds(s * 16, 16)
                        pr = (vb_v[p, sl] * vb_v[npairs + p, sl]
                              * vb_v[2 * npairs + p, sl]
                              * vb_v[3 * npairs + p, sl])
                        th = 1.0 - 2.0 / (jnp.exp(pr * two_c) + 1.0)
                        acc1[s] = acc1[s] + th
                for s in range(4):
                    s1b_v[nl, pl.ds(s * 16, 16)] = acc1[s]
                return 0

            lax.fori_loop(0, npc, nl_slow, 0)

        pltpu.sync_copy(s1b_v, s1_ref.at[pl.ds(nbase, npc)])
        pltpu.sync_copy(s2b_v, s2_ref.at[pl.ds(nbase, npc)])
        return 0

    lax.fori_loop(0, iters, it_body, 0)


def kernel(embedding, node_edges, edge_nodes, W_p, b_p, W_q, b_q,
           W_p2a, b_p2a, W_p2b, b_p2b, W_a, b_a):
    f32 = jnp.float32
    n, feat = embedding.shape
    d_deg = node_edges.shape[1]
    e, k = edge_nodes.shape
    rank = W_p.shape[1]
    out = W_q.shape[1]
    hid = W_p2a.shape[1]

    num = float(d_deg) ** (1.0 / float(k))
    two_c = 2.0 * num / float(math.factorial(k - 1))
    inv_d = 1.0 / float(d_deg)

    rank_p = 128
    rank_s = 64
    ctw = 256
    npc = 8
    n_pad = -(-n // (_NW * npc)) * (_NW * npc)
    ec = 128
    e_pad = -(-e // (_NW * ec)) * (_NW * ec)

    ne32 = node_edges.astype(jnp.int32)
    en32 = edge_nodes.astype(jnp.int32)
    x_pad = jnp.zeros((n_pad, feat), f32).at[:n].set(embedding.astype(f32))
    nef = jnp.zeros((n_pad, d_deg), jnp.int32).at[:n].set(ne32).reshape(-1)
    enf = jnp.zeros((e_pad, k), jnp.int32).at[:e].set(en32).reshape(-1)

    n_chunks = n_pad // npc
    hasc = jnp.any(
        en32[ne32] == jnp.arange(n, dtype=jnp.int32)[:, None, None],
        axis=(1, 2))
    hasc_p = jnp.zeros((n_pad,), jnp.bool_).at[:n].set(hasc)
    cflag = jnp.any(hasc_p.reshape(n_chunks, npc), axis=1).astype(jnp.int32)
    n_iters = n_chunks // _NW
    flags = jnp.zeros((_NW, 128), jnp.int32).at[:, :n_iters].set(
        cflag.reshape(_NW, n_iters)).reshape(-1)

    wa_x = W_a[:feat]
    ba_full = (b_a + W_a[feat]).reshape(1, out)
    wp_x = jnp.zeros((feat, rank_p), f32).at[:, :rank].set(num * W_p[:feat])
    bp_full = jnp.zeros((1, rank_p), f32).at[0, :rank].set(num * (b_p + W_p[feat]))
    w2a_x = W_p2a[:feat]
    b2a_full = (b_p2a + W_p2a[feat]).reshape(1, hid)
    b2b_full = b_p2b.reshape(1, out)
    wq_s = jnp.zeros((rank_s, out), f32).at[:rank].set(W_q * inv_d)
    bq_full = b_q.reshape(1, out)

    br = 512
    grid = (n_pad // br,)
    wspec = lambda a: pl.BlockSpec(a.shape, lambda i: (0, 0))
    res, tv, e2t = pl.pallas_call(
        _tc_proj_body,
        grid=grid,
        in_specs=[pl.BlockSpec((br, feat), lambda i: (i, 0)),
                  wspec(wa_x), wspec(ba_full), wspec(wp_x), wspec(bp_full),
                  wspec(w2a_x), wspec(b2a_full), wspec(w2b := W_p2b), wspec(b2b_full)],
        out_specs=[pl.BlockSpec((br, out), lambda i: (i, 0)),
                   pl.BlockSpec((br, rank_p), lambda i: (i, 0)),
                   pl.BlockSpec((br, out), lambda i: (i, 0))],
        out_shape=[jax.ShapeDtypeStruct((n_pad, out), f32),
                   jax.ShapeDtypeStruct((n_pad, rank_p), f32),
                   jax.ShapeDtypeStruct((n_pad, out), f32)],
    )(x_pad, wa_x, ba_full, wp_x, bp_full, w2a_x, b2a_full, w2b, b2b_full)

    tvb = jnp.concatenate([tv, jnp.ones((8, rank_p), f32)], axis=0)

    mesh = plsc.VectorSubcoreMesh(core_axis_name="c", subcore_axis_name="s")

    e_iters = e_pad // (_NW * ec)
    ct = pl.kernel(
        functools.partial(_edge_body, ec=ec, iters=e_iters, inv_d=inv_d,
                          two_c=two_c),
        out_type=jax.ShapeDtypeStruct((e_pad, ctw), f32),
        mesh=mesh,
        scratch_types=[
            pltpu.VMEM((ec * 4,), jnp.int32),
            pltpu.VMEM((ec * 4, out), f32),
            pltpu.VMEM((ec, ctw), f32),
            pltpu.SemaphoreType.DMA,
        ],
    )(enf, tvb, e2t)

    n_iters = n_pad // (_NW * npc)
    npairs = npc * d_deg
    s1, s2 = pl.kernel(
        functools.partial(_node_body, npc=npc, iters=n_iters, d_deg=d_deg,
                          two_c=two_c, ones_row=n_pad),
        out_type=(jax.ShapeDtypeStruct((n_pad, rank_s), f32),
                  jax.ShapeDtypeStruct((n_pad, out), f32)),
        mesh=mesh,
        scratch_types=[
            pltpu.VMEM((npairs,), jnp.int32),
            pltpu.VMEM((4, npairs), jnp.int32),
            pltpu.VMEM((4, npairs), jnp.int32),
            pltpu.VMEM((npairs, ctw), f32),
            pltpu.VMEM((npairs * 4, rank_p), f32),
            pltpu.VMEM((npc, rank_s), f32),
            pltpu.VMEM((npc, out), f32),
            pltpu.VMEM((128,), jnp.int32),
            pltpu.SemaphoreType.DMA,
            pltpu.SemaphoreType.DMA,
        ],
    )(nef, flags, en32[:, 0], en32[:, 1], en32[:, 2], en32[:, 3], tvb, ct)

    o = pl.pallas_call(
        _tc_out_body,
        grid=grid,
        in_specs=[pl.BlockSpec((br, rank_s), lambda i: (i, 0)),
                  pl.BlockSpec((br, out), lambda i: (i, 0)),
                  pl.BlockSpec((br, out), lambda i: (i, 0)),
                  wspec(wq_s), wspec(bq_full)],
        out_specs=pl.BlockSpec((br, out), lambda i: (i, 0)),
        out_shape=jax.ShapeDtypeStruct((n_pad, out), f32),
    )(s1, s2, res, wq_s, bq_full)

    return o[:n]
